# baseline (speedup 1.0000x reference)
import functools

import jax
import jax.numpy as jnp
from jax.experimental import pallas as pl

_MARGIN = 0.3
_N = 4096
_D = 512
_BLK = 512


def _loss_body(a_ref, b_ref, ti_ref, tj_ref, out_ref):
    i = pl.program_id(0)
    j = pl.program_id(1)

    @pl.when((i == 0) & (j == 0))
    def _init():
        out_ref[...] = jnp.zeros((1, 1), jnp.float32)

    sim = jax.lax.dot_general(
        a_ref[...],
        b_ref[...],
        dimension_numbers=(((1,), (1,)), ((), ())),
        preferred_element_type=jnp.float32,
    )
    same = ti_ref[...][:, None] == tj_ref[...][None, :]
    pos = jnp.where(same & (sim < 1.0), 1.0 - sim, 0.0)
    neg = jnp.where(jnp.logical_not(same) & (sim > _MARGIN), sim, 0.0)
    out_ref[...] += jnp.sum(pos + neg).reshape(1, 1)


@functools.partial(jax.jit, static_argnames=())
def kernel(inputs, targets):
    t32 = targets.astype(jnp.int32)
    nblk = _N // _BLK
    total = pl.pallas_call(
        _loss_body,
        grid=(nblk, nblk),
        in_specs=[
            pl.BlockSpec((_BLK, _D), lambda i, j: (i, 0)),
            pl.BlockSpec((_BLK, _D), lambda i, j: (j, 0)),
            pl.BlockSpec((_BLK,), lambda i, j: (i,)),
            pl.BlockSpec((_BLK,), lambda i, j: (j,)),
        ],
        out_specs=pl.BlockSpec((1, 1), lambda i, j: (0, 0)),
        out_shape=jax.ShapeDtypeStruct((1, 1), jnp.float32),
    )(inputs, inputs, t32, t32)
    return total[0, 0] / _N


# speedup vs baseline: 1.0865x; 1.0865x over previous
import functools

import jax
import jax.numpy as jnp
from jax.experimental import pallas as pl

_MARGIN = 0.3
_N = 4096
_D = 512
_BLK = 512


def _loss_body(a_ref, b_ref, ti_ref, tj_ref, out_ref):
    i = pl.program_id(0)
    j = pl.program_id(1)

    @pl.when((i == 0) & (j == 0))
    def _init():
        out_ref[...] = jnp.zeros((1, 1), jnp.float32)

    sim = jax.lax.dot_general(
        a_ref[...],
        b_ref[...],
        dimension_numbers=(((1,), (1,)), ((), ())),
        preferred_element_type=jnp.float32,
    )
    same = ti_ref[...][:, None] == tj_ref[...][None, :]
    pos = jnp.maximum(1.0 - sim, 0.0)
    neg = jnp.where(sim > _MARGIN, sim, 0.0)
    out_ref[...] += jnp.sum(jnp.where(same, pos, neg)).reshape(1, 1)

    nblk = _N // _BLK
    @pl.when((i == nblk - 1) & (j == nblk - 1))
    def _finish():
        out_ref[...] *= 1.0 / _N


@functools.partial(jax.jit, static_argnames=())
def kernel(inputs, targets):
    t32 = targets.astype(jnp.int32)
    nblk = _N // _BLK
    total = pl.pallas_call(
        _loss_body,
        grid=(nblk, nblk),
        in_specs=[
            pl.BlockSpec((_BLK, _D), lambda i, j: (i, 0)),
            pl.BlockSpec((_BLK, _D), lambda i, j: (j, 0)),
            pl.BlockSpec((_BLK,), lambda i, j: (i,)),
            pl.BlockSpec((_BLK,), lambda i, j: (j,)),
        ],
        out_specs=pl.BlockSpec((1, 1), lambda i, j: (0, 0)),
        out_shape=jax.ShapeDtypeStruct((1, 1), jnp.float32),
    )(inputs, inputs, t32, t32)
    return total[0, 0]
